# baseline (speedup 1.0000x reference)
import functools

import jax
import jax.numpy as jnp
from jax import lax
from jax.experimental import pallas as pl
from jax.experimental.pallas import tpu as pltpu
from jax.experimental.pallas import tpu_sc as plsc

V = 1_000_000
H = 64
T = 200
SQ = 16384
NC, NS = 2, 16
NW = NC * NS
SPW = SQ // NW
SPLIT = (0, 128), (128, 128)


def _make_gather():
    mesh = plsc.VectorSubcoreMesh(core_axis_name="c", subcore_axis_name="s")

    @functools.partial(
        pl.kernel,
        mesh=mesh,
        out_type=jax.ShapeDtypeStruct((SQ, T, 128), jnp.float32),
        scratch_types=[
            pltpu.VMEM((2, 8, 256), jnp.int32),
            pltpu.VMEM((3, 256, 128), jnp.float32),
            pltpu.SemaphoreType.DMA,
            pltpu.SemaphoreType.DMA,
        ],
        compiler_params=pltpu.CompilerParams(
            use_tc_tiling_on_sc=True, needs_layout_passes=False
        ),
    )
    def gather_kernel(tbl_hbm, x_hbm, out_hbm, idxb, rows, semg, semo):
        core = lax.axis_index("c")
        sid = lax.axis_index("s")
        wid = sid * NC + core
        s0 = wid * SPW

        def fire(u):
            @pl.when(lax.rem(u, 8) == 0)
            def _():
                pltpu.sync_copy(
                    x_hbm.at[pl.ds(pl.multiple_of(s0 + u, 8), 8)],
                    idxb.at[lax.rem(lax.div(u, 8), 2)],
                )
            for off, ln in SPLIT:
                pltpu.async_copy(
                    tbl_hbm.at[
                        idxb.at[lax.rem(lax.div(u, 8), 2), lax.rem(u, 8),
                                pl.ds(off, ln)]
                    ],
                    rows.at[lax.rem(u, 3), pl.ds(off, ln)], semg,
                )

        def wait_gather(b):
            pltpu.make_async_copy(
                tbl_hbm.at[pl.ds(0, 256)], rows.at[b], semg
            ).wait()

        def wait_out():
            pltpu.make_async_copy(
                rows.at[0, pl.ds(0, T)], out_hbm.at[0], semo
            ).wait()

        fire(0)
        fire(1)

        def step(u, _):
            @pl.when(u >= 4)
            def _():
                wait_out()

            wait_gather(lax.rem(u - 2, 3))

            @pl.when(u < SPW)
            def _():
                fire(u)

            pltpu.async_copy(
                rows.at[lax.rem(u - 2, 3), pl.ds(0, T)],
                out_hbm.at[s0 + u - 2], semo,
            )
            return _

        lax.fori_loop(2, SPW + 2, step, None)

        for _ in range(2):
            wait_out()

    return gather_kernel


_gather = _make_gather()


def kernel(x, weight):
    tbl = jnp.pad(weight, ((0, 0), (0, 128 - H)))
    xpad = jnp.pad(x.astype(jnp.int32), ((0, 0), (0, 256 - T)))
    out128 = _gather(tbl, xpad)
    return out128[:, :, :H]
